# Initial kernel scaffold; baseline (speedup 1.0000x reference)
import jax
import jax.numpy as jnp
from jax.experimental import pallas as pl


def kernel(inputs):
    raise NotImplementedError("write your pallas kernel here")


# speedup vs baseline: 1.9016x; 1.9016x over previous
import functools

import jax
import jax.numpy as jnp
from jax.experimental import pallas as pl

_EPS = 1e-2
_NUM_ITERS = 10


def _softranks_body(x_ref, o_ref, *, n: int):
    inv_eps = 1.0 / _EPS
    x = x_ref[...].reshape(1, n)
    xmin = jnp.min(x)
    xmax = jnp.max(x)
    z_row = (x - xmin) / (xmax - xmin + 1e-12)
    z_col = z_row.reshape(n, 1)
    iota_row = jax.lax.broadcasted_iota(jnp.int32, (1, n), 1).astype(jnp.float32)
    y_row = iota_row * (1.0 / (n - 1))
    c = (z_col - y_row) ** 2 * inv_eps
    log_w = -jnp.log(jnp.float32(n))

    def step(_, carry):
        f_col, _ = carry
        m1_arg = f_col * inv_eps - c
        m1 = jnp.max(m1_arg, axis=0, keepdims=True)
        s1 = jnp.sum(jnp.exp(m1_arg - m1), axis=0, keepdims=True)
        g_row = _EPS * (log_w - (jnp.log(s1) + m1))
        m2_arg = g_row * inv_eps - c
        m2 = jnp.max(m2_arg, axis=1, keepdims=True)
        s2 = jnp.sum(jnp.exp(m2_arg - m2), axis=1, keepdims=True)
        f_col = _EPS * (log_w - (jnp.log(s2) + m2))
        return f_col, g_row

    f0 = jnp.zeros((n, 1), jnp.float32)
    g0 = jnp.zeros((1, n), jnp.float32)
    f_col, g_row = jax.lax.fori_loop(0, _NUM_ITERS, step, (f0, g0))

    p = jnp.exp(f_col * inv_eps + g_row * inv_eps - c)
    w_row = (iota_row + 1.0) * jnp.float32(n)
    ranks_col = jnp.sum(p * w_row, axis=1, keepdims=True) - 1.0
    o_ref[...] = ranks_col.reshape(1, 1, n)


@jax.jit
def kernel(inputs):
    b, n = inputs.shape
    out = pl.pallas_call(
        functools.partial(_softranks_body, n=n),
        grid=(b,),
        in_specs=[pl.BlockSpec((1, 1, n), lambda i: (i, 0, 0))],
        out_specs=pl.BlockSpec((1, 1, n), lambda i: (i, 0, 0)),
        out_shape=jax.ShapeDtypeStruct((b, 1, n), jnp.float32),
    )(inputs.reshape(b, 1, n))
    return out.reshape(b, n)


# speedup vs baseline: 2.3323x; 1.2265x over previous
import functools
import math

import jax
import jax.numpy as jnp
from jax.experimental import pallas as pl

_EPS = 1e-2
_NUM_ITERS = 10
_SCALE = 1.0 / (_EPS * math.log(2.0))
_SQRT_SCALE = math.sqrt(_SCALE)
_SHIFT = 44.0


def _softranks_body(x_ref, o_ref, *, n: int):
    x = x_ref[...].reshape(1, n)
    xmin = jnp.min(x)
    xmax = jnp.max(x)
    z_row = (x - xmin) * (_SQRT_SCALE / (xmax - xmin + 1e-12))
    iota_row = jax.lax.broadcasted_iota(jnp.int32, (1, n), 1).astype(jnp.float32)
    y_row = iota_row * (_SQRT_SCALE / (n - 1))
    z_col = z_row.reshape(n, 1)
    c = (z_col - y_row) ** 2
    log2n = math.log2(n)

    def step(_, carry):
        phi_col, _ = carry
        u1 = jnp.max(phi_col) - _SHIFT
        s1 = jnp.sum(jnp.exp2((phi_col - u1) - c), axis=0, keepdims=True)
        gamma_row = (-log2n - u1) - jnp.log2(s1)
        u2 = jnp.max(gamma_row) - _SHIFT
        s2 = jnp.sum(jnp.exp2((gamma_row - u2) - c), axis=1, keepdims=True)
        phi_col = (-log2n - u2) - jnp.log2(s2)
        return phi_col, gamma_row

    phi0 = jnp.zeros((n, 1), jnp.float32)
    g0 = jnp.zeros((1, n), jnp.float32)
    phi_col, gamma_row = jax.lax.fori_loop(0, _NUM_ITERS, step, (phi0, g0))

    p = jnp.exp2(phi_col + (gamma_row - c))
    w_row = (iota_row + 1.0) * jnp.float32(n)
    ranks_col = jnp.sum(p * w_row, axis=1, keepdims=True) - 1.0
    o_ref[...] = ranks_col.reshape(1, 1, n)


@jax.jit
def kernel(inputs):
    b, n = inputs.shape
    out = pl.pallas_call(
        functools.partial(_softranks_body, n=n),
        grid=(b,),
        in_specs=[pl.BlockSpec((1, 1, n), lambda i: (i, 0, 0))],
        out_specs=pl.BlockSpec((1, 1, n), lambda i: (i, 0, 0)),
        out_shape=jax.ShapeDtypeStruct((b, 1, n), jnp.float32),
    )(inputs.reshape(b, 1, n))
    return out.reshape(b, n)
